# baseline (speedup 1.0000x reference)
import jax
import jax.numpy as jnp
from jax import lax
from jax.experimental import pallas as pl
from jax.experimental.pallas import tpu as pltpu
from jax.experimental.pallas import tpu_sc as plsc

N, D_MODEL, D_FF, C = 4096, 2048, 4096, 2
BLK_N = 1024
BLK_F = 512
NF = D_FF // BLK_F
R = N // BLK_N

NC, NS, L = 2, 16, 16
NW = NC * NS
RPW = N // NW
CPW = RPW // L
GCH = 32
NGC = RPW // GCH

_sc_mesh = plsc.VectorSubcoreMesh(core_axis_name="c", subcore_axis_name="s")


def _route_kernel(mask_hbm, x_hbm, xg_hbm, pos_hbm, kcnt_hbm,
                  mv, idxv, posv, dstm, rows, kv, gsem, ssem):
    wid = lax.axis_index("s") * NC + lax.axis_index("c")
    base = wid * RPW

    pltpu.sync_copy(mask_hbm, mv)

    def _psum(c, acc):
        return acc + jnp.sum(mv[pl.ds(c * L, L)])
    tile_off = lax.fori_loop(0, wid * CPW, _psum, jnp.int32(0))
    k_total = lax.fori_loop(0, N // L, _psum, jnp.int32(0))

    zero16 = jnp.zeros((L,), jnp.int32)
    for c in range(CPW):
        idxv[pl.ds(c * L, L)] = zero16
    run = jnp.int32(0)
    io = lax.iota(jnp.int32, L)
    for c in range(CPW):
        m16 = mv[pl.ds(base + c * L, L)]
        incl = plsc.cumsum(m16) + run
        posv[pl.ds(c * L, L)] = incl + tile_off - 1
        mb = m16 != 0
        gids = io + (base + c * L)
        plsc.store_scatter(idxv, [incl - 1], gids, mask=mb)
        run = run + jnp.sum(m16)
    local_cnt = run

    for j in range(NGC):
        for h in range(GCH // L):
            lane = io + (j * GCH + h * L)
            d = jnp.where(lane < local_cnt, tile_off + lane, N - 1)
            dstm[j, pl.ds(h * L, L)] = d

    pltpu.sync_copy(posv, pos_hbm.at[pl.ds(base, RPW)])

    @pl.when(wid == 0)
    def _():
        kv[...] = jnp.full((L,), k_total, jnp.int32)
        pltpu.sync_copy(kv, kcnt_hbm)

    for j in range(NGC):
        @pl.when(j * GCH < local_cnt)
        def _():
            pltpu.async_copy(x_hbm.at[idxv.at[pl.ds(j * GCH, GCH)]],
                             rows, gsem).wait()
            pltpu.async_copy(rows, xg_hbm.at[dstm.at[j]], ssem).wait()


def _route(mask_i32, x):
    return pl.kernel(
        _route_kernel,
        out_type=[
            jax.ShapeDtypeStruct((N, D_MODEL), jnp.float32),
            jax.ShapeDtypeStruct((N,), jnp.int32),
            jax.ShapeDtypeStruct((L,), jnp.int32),
        ],
        mesh=_sc_mesh,
        compiler_params=pltpu.CompilerParams(needs_layout_passes=False),
        scratch_types=[
            pltpu.VMEM((N,), jnp.int32),
            pltpu.VMEM((RPW,), jnp.int32),
            pltpu.VMEM((RPW,), jnp.int32),
            pltpu.VMEM((NGC, GCH), jnp.int32),
            pltpu.VMEM((GCH, D_MODEL), jnp.float32),
            pltpu.VMEM((L,), jnp.int32),
            pltpu.SemaphoreType.DMA,
            pltpu.SemaphoreType.DMA,
        ],
    )(mask_i32, x)


def _dense_kernel(x_ref, m_ref,
                  w1a1_ref, w2a1_ref, w1b1_ref, w2b1_ref,
                  w1a2_ref, w2a2_ref, w1b2_ref, w2b2_ref,
                  out1_ref, out2_ref, pred_ref):
    f = pl.program_id(1)
    xb = x_ref[...]

    def stream(w1_ref, w2_ref):
        h = jax.nn.gelu(jnp.dot(xb, w1_ref[...],
                                preferred_element_type=jnp.float32))
        return jnp.dot(h, w2_ref[...], preferred_element_type=jnp.float32)

    c1 = stream(w1a1_ref, w2a1_ref) + stream(w1b1_ref, w2b1_ref)
    c2 = stream(w1a2_ref, w2a2_ref) + stream(w1b2_ref, w2b2_ref)

    @pl.when(f == 0)
    def _():
        out1_ref[...] = c1
        out2_ref[...] = c2

    @pl.when(f > 0)
    def _():
        out1_ref[...] += c1
        out2_ref[...] += c2

    @pl.when(f == NF - 1)
    def _():
        l1 = out1_ref[...]
        mask = m_ref[...] != 0
        out2_ref[...] = jnp.where(mask, out2_ref[...], 0.0)
        pred_ref[...] = (l1[:, 1:2] > l1[:, 0:1]).astype(jnp.float32)


def _dense(x, mask_i32, l1_W1a, l1_W2a, l1_W1b, l1_W2b,
           l2_W1a, l2_W2a, l2_W1b, l2_W2b):
    row_blk = lambda i, f: (i, 0)
    w1_blk = lambda i, f: (0, f)
    w2_blk = lambda i, f: (f, 0)
    return pl.pallas_call(
        _dense_kernel,
        grid=(R, NF),
        compiler_params=pltpu.CompilerParams(
            vmem_limit_bytes=64 * 1024 * 1024,
            dimension_semantics=("parallel", "arbitrary")),
        in_specs=[
            pl.BlockSpec((BLK_N, D_MODEL), row_blk),
            pl.BlockSpec((BLK_N, 1), row_blk),
            pl.BlockSpec((D_MODEL, BLK_F), w1_blk),
            pl.BlockSpec((BLK_F, C), w2_blk),
            pl.BlockSpec((D_MODEL, BLK_F), w1_blk),
            pl.BlockSpec((BLK_F, C), w2_blk),
            pl.BlockSpec((D_MODEL, BLK_F), w1_blk),
            pl.BlockSpec((BLK_F, C), w2_blk),
            pl.BlockSpec((D_MODEL, BLK_F), w1_blk),
            pl.BlockSpec((BLK_F, C), w2_blk),
        ],
        out_specs=[
            pl.BlockSpec((BLK_N, C), row_blk),
            pl.BlockSpec((BLK_N, C), row_blk),
            pl.BlockSpec((BLK_N, 1), row_blk),
        ],
        out_shape=[
            jax.ShapeDtypeStruct((N, C), jnp.float32),
            jax.ShapeDtypeStruct((N, C), jnp.float32),
            jax.ShapeDtypeStruct((N, 1), jnp.float32),
        ],
    )(x, mask_i32, l1_W1a, l1_W2a, l1_W1b, l1_W2b,
      l2_W1a, l2_W2a, l2_W1b, l2_W2b)


def _l1_kernel(x_ref, w1a_ref, w2a_ref, w1b_ref, w2b_ref, out1_ref, pred_ref):
    f = pl.program_id(1)
    xb = x_ref[...]

    def stream(w1_ref, w2_ref):
        h = jax.nn.gelu(jnp.dot(xb, w1_ref[...],
                                preferred_element_type=jnp.float32))
        return jnp.dot(h, w2_ref[...], preferred_element_type=jnp.float32)

    c1 = stream(w1a_ref, w2a_ref) + stream(w1b_ref, w2b_ref)

    @pl.when(f == 0)
    def _():
        out1_ref[...] = c1

    @pl.when(f > 0)
    def _():
        out1_ref[...] += c1

    @pl.when(f == NF - 1)
    def _():
        l1 = out1_ref[...]
        pred_ref[...] = (l1[:, 1:2] > l1[:, 0:1]).astype(jnp.float32)


def _layer1(x, w1a, w2a, w1b, w2b):
    row_blk = lambda i, f: (i, 0)
    w1_blk = lambda i, f: (0, f)
    w2_blk = lambda i, f: (f, 0)
    return pl.pallas_call(
        _l1_kernel,
        grid=(R, NF),
        in_specs=[
            pl.BlockSpec((BLK_N, D_MODEL), row_blk),
            pl.BlockSpec((D_MODEL, BLK_F), w1_blk),
            pl.BlockSpec((BLK_F, C), w2_blk),
            pl.BlockSpec((D_MODEL, BLK_F), w1_blk),
            pl.BlockSpec((BLK_F, C), w2_blk),
        ],
        out_specs=[
            pl.BlockSpec((BLK_N, C), row_blk),
            pl.BlockSpec((BLK_N, 1), row_blk),
        ],
        out_shape=[
            jax.ShapeDtypeStruct((N, C), jnp.float32),
            jax.ShapeDtypeStruct((N, 1), jnp.float32),
        ],
    )(x, w1a, w2a, w1b, w2b)


def _l2_kernel(kref, xg_ref, w1a_ref, w2a_ref, w1b_ref, w2b_ref, l2c_ref):
    i = pl.program_id(0)
    f = pl.program_id(1)
    nab = (kref[0] + BLK_N - 1) // BLK_N

    @pl.when(i < nab)
    def _():
        xb = xg_ref[...]

        def stream(w1_ref, w2_ref):
            h = jax.nn.gelu(jnp.dot(xb, w1_ref[...],
                                    preferred_element_type=jnp.float32))
            return jnp.dot(h, w2_ref[...], preferred_element_type=jnp.float32)

        c2 = stream(w1a_ref, w2a_ref) + stream(w1b_ref, w2b_ref)

        @pl.when(f == 0)
        def _():
            l2c_ref[...] = c2

        @pl.when(f > 0)
        def _():
            l2c_ref[...] += c2


def _nab(kref):
    return (kref[0] + BLK_N - 1) // BLK_N


def _layer2(kcnt, xg, w1a, w2a, w1b, w2b):
    def xg_blk(i, f, kref):
        return (jnp.minimum(i, jnp.maximum(_nab(kref) - 1, 0)), 0)

    def w1_blk(i, f, kref):
        return (0, jnp.where(i < _nab(kref), f, NF - 1))

    def w2_blk(i, f, kref):
        return (jnp.where(i < _nab(kref), f, NF - 1), 0)

    return pl.pallas_call(
        _l2_kernel,
        grid_spec=pltpu.PrefetchScalarGridSpec(
            num_scalar_prefetch=1,
            grid=(R, NF),
            in_specs=[
                pl.BlockSpec((BLK_N, D_MODEL), xg_blk),
                pl.BlockSpec((D_MODEL, BLK_F), w1_blk),
                pl.BlockSpec((BLK_F, C), w2_blk),
                pl.BlockSpec((D_MODEL, BLK_F), w1_blk),
                pl.BlockSpec((BLK_F, C), w2_blk),
            ],
            out_specs=pl.BlockSpec((BLK_N, C), lambda i, f, kref: (i, 0)),
        ),
        out_shape=jax.ShapeDtypeStruct((N, C), jnp.float32),
    )(kcnt, xg, w1a, w2a, w1b, w2b)


def _unroute_kernel(l2f_hbm, pos_hbm, mask_hbm, out_hbm, l2v, pv, mv, outv):
    wid = lax.axis_index("s") * NC + lax.axis_index("c")
    base = wid * RPW

    pltpu.sync_copy(l2f_hbm, l2v)
    pltpu.sync_copy(pos_hbm.at[pl.ds(base, RPW)], pv)
    pltpu.sync_copy(mask_hbm.at[pl.ds(base, RPW)], mv)

    io = lax.iota(jnp.int32, L)
    zf = jnp.zeros((L,), jnp.float32)
    for c in range(CPW):
        p = jnp.maximum(pv[pl.ds(c * L, L)], 0)
        m = mv[pl.ds(c * L, L)] != 0
        g0 = plsc.load_gather(l2v, [2 * p])
        g1 = plsc.load_gather(l2v, [2 * p + 1])
        r0 = jnp.where(m, g0, zf)
        r1 = jnp.where(m, g1, zf)
        plsc.store_scatter(outv, [c * 2 * L + 2 * io], r0)
        plsc.store_scatter(outv, [c * 2 * L + 2 * io + 1], r1)

    pltpu.sync_copy(outv, out_hbm.at[pl.ds(base * 2, RPW * 2)])


def _unroute(l2flat, pos, mask_i32):
    return pl.kernel(
        _unroute_kernel,
        out_type=jax.ShapeDtypeStruct((2 * N,), jnp.float32),
        mesh=_sc_mesh,
        compiler_params=pltpu.CompilerParams(needs_layout_passes=False),
        scratch_types=[
            pltpu.VMEM((2 * N,), jnp.float32),
            pltpu.VMEM((RPW,), jnp.int32),
            pltpu.VMEM((RPW,), jnp.int32),
            pltpu.VMEM((2 * RPW,), jnp.float32),
        ],
    )(l2flat, pos, mask_i32)


B2N = 512
R2 = N // B2N


def _twophase_kernel(kref, x_ref, xg_ref,
                     w1a1_ref, w2a1_ref, w1b1_ref, w2b1_ref,
                     w1a2_ref, w2a2_ref, w1b2_ref, w2b2_ref,
                     out1_ref, pred_ref, l2c_ref):
    t = pl.program_id(0)
    f = pl.program_id(1)
    nab = (kref[0] + B2N - 1) // B2N

    def stream(xb, w1_ref, w2_ref):
        h = jax.nn.gelu(jnp.dot(xb, w1_ref[...],
                                preferred_element_type=jnp.float32))
        return jnp.dot(h, w2_ref[...], preferred_element_type=jnp.float32)

    @pl.when(t < R)
    def _():
        xb = x_ref[...]
        c1 = stream(xb, w1a1_ref, w2a1_ref) + stream(xb, w1b1_ref, w2b1_ref)

        @pl.when(f == 0)
        def _():
            out1_ref[...] = c1

        @pl.when(f > 0)
        def _():
            out1_ref[...] += c1

        @pl.when(f == NF - 1)
        def _():
            l1 = out1_ref[...]
            pred_ref[...] = (l1[:, 1:2] > l1[:, 0:1]).astype(jnp.float32)

    @pl.when((t >= R) & (t - R < nab))
    def _():
        xb = xg_ref[...]
        c2 = stream(xb, w1a2_ref, w2a2_ref) + stream(xb, w1b2_ref, w2b2_ref)

        @pl.when(f == 0)
        def _():
            l2c_ref[...] = c2

        @pl.when(f > 0)
        def _():
            l2c_ref[...] += c2


def _twophase(kcnt, x, xg,
              l1_W1a, l1_W2a, l1_W1b, l1_W2b,
              l2_W1a, l2_W2a, l2_W1b, l2_W2b):
    def nab_of(kref):
        return (kref[0] + B2N - 1) // B2N

    def x_blk(t, f, kref):
        return (jnp.minimum(t, R - 1), 0)

    def xg_blk(t, f, kref):
        t2 = jnp.maximum(t - R, 0)
        return (jnp.minimum(t2, jnp.maximum(nab_of(kref) - 1, 0)), 0)

    def w1l1_blk(t, f, kref):
        return (0, jnp.where(t < R, f, NF - 1))

    def w2l1_blk(t, f, kref):
        return (jnp.where(t < R, f, NF - 1), 0)

    def w1l2_blk(t, f, kref):
        return (0, jnp.where((t >= R) & (t - R < nab_of(kref)), f, 0))

    def w2l2_blk(t, f, kref):
        return (jnp.where((t >= R) & (t - R < nab_of(kref)), f, 0), 0)

    def out1_blk(t, f, kref):
        return (jnp.minimum(t, R - 1), 0)

    def l2c_blk(t, f, kref):
        return (jnp.minimum(jnp.maximum(t - R, 0), R2 - 1), 0)

    return pl.pallas_call(
        _twophase_kernel,
        grid_spec=pltpu.PrefetchScalarGridSpec(
            num_scalar_prefetch=1,
            grid=(R + R2, NF),
            in_specs=[
                pl.BlockSpec((BLK_N, D_MODEL), x_blk),
                pl.BlockSpec((B2N, D_MODEL), xg_blk),
                pl.BlockSpec((D_MODEL, BLK_F), w1l1_blk),
                pl.BlockSpec((BLK_F, C), w2l1_blk),
                pl.BlockSpec((D_MODEL, BLK_F), w1l1_blk),
                pl.BlockSpec((BLK_F, C), w2l1_blk),
                pl.BlockSpec((D_MODEL, BLK_F), w1l2_blk),
                pl.BlockSpec((BLK_F, C), w2l2_blk),
                pl.BlockSpec((D_MODEL, BLK_F), w1l2_blk),
                pl.BlockSpec((BLK_F, C), w2l2_blk),
            ],
            out_specs=[
                pl.BlockSpec((BLK_N, C), out1_blk),
                pl.BlockSpec((BLK_N, 1), out1_blk),
                pl.BlockSpec((B2N, C), l2c_blk),
            ],
        ),
        out_shape=[
            jax.ShapeDtypeStruct((N, C), jnp.float32),
            jax.ShapeDtypeStruct((N, 1), jnp.float32),
            jax.ShapeDtypeStruct((N, C), jnp.float32),
        ],
        compiler_params=pltpu.CompilerParams(
            vmem_limit_bytes=100 * 1024 * 1024),
    )(kcnt, x, xg,
      l1_W1a, l1_W2a, l1_W1b, l1_W2b,
      l2_W1a, l2_W2a, l2_W1b, l2_W2b)


XCH = 128
NXCH = N // XCH


def _resident_kernel(x_hbm, m_ref,
                     w1a1_ref, w1b1_ref, w1a2_ref, w1b2_ref, w2c_ref,
                     out_ref,
                     xs, stage, acc, sem):
    f = pl.program_id(0)
    j = pl.program_id(1)

    @pl.when((f == 0) & (j == 0))
    def _():
        def body(k, carry):
            dma = pltpu.make_async_copy(x_hbm.at[pl.ds(k * XCH, XCH)],
                                        stage, sem)
            dma.start()
            dma.wait()
            xs[pl.ds(k * XCH, XCH), :] = stage[...].astype(jnp.bfloat16)
            return carry
        lax.fori_loop(0, NXCH, body, 0)

    sl = pl.ds(j * BLK_N, BLK_N)
    xb = xs[sl, :]
    w2t = w2c_ref[...]

    def stream(w1_ref, w2):
        h = jax.nn.gelu(jnp.dot(xb, w1_ref[...].astype(jnp.bfloat16),
                                preferred_element_type=jnp.float32))
        return jnp.dot(h.astype(jnp.bfloat16), w2,
                       preferred_element_type=jnp.float32)

    c1 = stream(w1a1_ref, w2t[:, 0:2]) + stream(w1b1_ref, w2t[:, 2:4])
    c2 = stream(w1a2_ref, w2t[:, 4:6]) + stream(w1b2_ref, w2t[:, 6:8])
    c = jnp.concatenate([c1, c2], axis=1)

    @pl.when(f == 0)
    def _():
        acc[sl, :] = c

    @pl.when(f > 0)
    def _():
        acc[sl, :] += c

    @pl.when(f == NF - 1)
    def _():
        a = acc[sl, :]
        l1 = a[:, 0:2]
        mask = m_ref[...] != 0
        l2m = jnp.where(mask, a[:, 2:4], 0.0)
        pred = (l1[:, 1:2] > l1[:, 0:1]).astype(jnp.float32)
        out_ref[...] = jnp.concatenate(
            [l1, l2m, pred, jnp.zeros_like(pred)], axis=1)


def _resident(x, mask_i32, l1_W1a, l1_W2a, l1_W1b, l1_W2b,
              l2_W1a, l2_W2a, l2_W1b, l2_W2b):
    w2c = jnp.concatenate([l1_W2a, l1_W2b, l2_W2a, l2_W2b], axis=1)
    row_blk = lambda f, j: (j, 0)
    w1_blk = lambda f, j: (0, f)
    w2_blk = lambda f, j: (f, 0)
    return pl.pallas_call(
        _resident_kernel,
        grid=(NF, R),
        compiler_params=pltpu.CompilerParams(
            vmem_limit_bytes=64 * 1024 * 1024,
            dimension_semantics=("arbitrary", "arbitrary")),
        in_specs=[
            pl.BlockSpec(memory_space=pl.ANY),
            pl.BlockSpec((BLK_N, 1), row_blk),
            pl.BlockSpec((D_MODEL, BLK_F), w1_blk),
            pl.BlockSpec((D_MODEL, BLK_F), w1_blk),
            pl.BlockSpec((D_MODEL, BLK_F), w1_blk),
            pl.BlockSpec((D_MODEL, BLK_F), w1_blk),
            pl.BlockSpec((BLK_F, 8), w2_blk),
        ],
        out_specs=pl.BlockSpec((BLK_N, 6), row_blk),
        out_shape=jax.ShapeDtypeStruct((N, 6), jnp.float32),
        scratch_shapes=[
            pltpu.VMEM((N, D_MODEL), jnp.bfloat16),
            pltpu.VMEM((XCH, D_MODEL), jnp.float32),
            pltpu.VMEM((N, 4), jnp.float32),
            pltpu.SemaphoreType.DMA,
        ],
    )(x, mask_i32, l1_W1a, l1_W1b, l2_W1a, l2_W1b, w2c)


def kernel(x, mask, l1_W1a, l1_W2a, l1_W1b, l1_W2b,
           l2_W1a, l2_W2a, l2_W1b, l2_W2b):
    mask_i32 = mask.astype(jnp.int32)
    o = _resident(x, mask_i32.reshape(N, 1),
                  l1_W1a, l1_W2a, l1_W1b, l1_W2b,
                  l2_W1a, l2_W2a, l2_W1b, l2_W2b)
    out1 = o[:, 0:2]
    out2 = o[:, 2:4]
    pred = o[:, 4] > 0.5
    return (out1, out2, pred)


# speedup vs baseline: 1.0963x; 1.0963x over previous
import jax
import jax.numpy as jnp
from jax.experimental import pallas as pl
from jax.experimental.pallas import tpu as pltpu

N, D_MODEL, D_FF, C = 4096, 2048, 4096, 2
BLK_N = 1024
BLK_F = 512
NF = D_FF // BLK_F
R = N // BLK_N


def _mlp_kernel(x_ref, m_ref,
                w1a1_ref, w2a1_ref, w1b1_ref, w2b1_ref,
                w1a2_ref, w2a2_ref, w1b2_ref, w2b2_ref,
                out1_ref, out2_ref, pred_ref):
    f = pl.program_id(1)
    xb = x_ref[...]

    def stream(w1_ref, w2_ref):
        h = jax.nn.gelu(jnp.dot(xb, w1_ref[...],
                                preferred_element_type=jnp.float32))
        return jnp.dot(h, w2_ref[...], preferred_element_type=jnp.float32)

    c1 = stream(w1a1_ref, w2a1_ref) + stream(w1b1_ref, w2b1_ref)
    c2 = stream(w1a2_ref, w2a2_ref) + stream(w1b2_ref, w2b2_ref)

    @pl.when(f == 0)
    def _():
        out1_ref[...] = c1
        out2_ref[...] = c2

    @pl.when(f > 0)
    def _():
        out1_ref[...] += c1
        out2_ref[...] += c2

    @pl.when(f == NF - 1)
    def _():
        l1 = out1_ref[...]
        mask = m_ref[...] != 0
        out2_ref[...] = jnp.where(mask, out2_ref[...], 0.0)
        pred_ref[...] = (l1[:, 1:2] > l1[:, 0:1]).astype(jnp.float32)


def _dense(x, mask_i32, l1_W1a, l1_W2a, l1_W1b, l1_W2b,
           l2_W1a, l2_W2a, l2_W1b, l2_W2b):
    row_blk = lambda i, f: (i, 0)
    w1_blk = lambda i, f: (0, f)
    w2_blk = lambda i, f: (f, 0)
    return pl.pallas_call(
        _mlp_kernel,
        grid=(R, NF),
        compiler_params=pltpu.CompilerParams(
            vmem_limit_bytes=64 * 1024 * 1024,
            dimension_semantics=("parallel", "arbitrary")),
        in_specs=[
            pl.BlockSpec((BLK_N, D_MODEL), row_blk),
            pl.BlockSpec((BLK_N, 1), row_blk),
            pl.BlockSpec((D_MODEL, BLK_F), w1_blk),
            pl.BlockSpec((BLK_F, C), w2_blk),
            pl.BlockSpec((D_MODEL, BLK_F), w1_blk),
            pl.BlockSpec((BLK_F, C), w2_blk),
            pl.BlockSpec((D_MODEL, BLK_F), w1_blk),
            pl.BlockSpec((BLK_F, C), w2_blk),
            pl.BlockSpec((D_MODEL, BLK_F), w1_blk),
            pl.BlockSpec((BLK_F, C), w2_blk),
        ],
        out_specs=[
            pl.BlockSpec((BLK_N, C), row_blk),
            pl.BlockSpec((BLK_N, C), row_blk),
            pl.BlockSpec((BLK_N, 1), row_blk),
        ],
        out_shape=[
            jax.ShapeDtypeStruct((N, C), jnp.float32),
            jax.ShapeDtypeStruct((N, C), jnp.float32),
            jax.ShapeDtypeStruct((N, 1), jnp.float32),
        ],
    )(x, mask_i32, l1_W1a, l1_W2a, l1_W1b, l1_W2b,
      l2_W1a, l2_W2a, l2_W1b, l2_W2b)


def kernel(x, mask, l1_W1a, l1_W2a, l1_W1b, l1_W2b,
           l2_W1a, l2_W2a, l2_W1b, l2_W2b):
    mask_i32 = mask.astype(jnp.int32).reshape(N, 1)
    out1, out2, predf = _dense(x, mask_i32, l1_W1a, l1_W2a, l1_W1b, l1_W2b,
                               l2_W1a, l2_W2a, l2_W1b, l2_W2b)
    pred = predf.reshape(N).astype(jnp.bool_)
    return (out1, out2, pred)
